# baseline (speedup 1.0000x reference)
import jax
import jax.numpy as jnp
from jax import lax
from jax.experimental import pallas as pl
from jax.experimental.pallas import tpu as pltpu
from jax.experimental.pallas import tpu_sc as plsc

N_USER = 25000
N_ITEM = 25000
N = N_USER + N_ITEM
E = 800000
D = 64
N_LAYERS = 3

NC = 2
NS = 16
HALF = 32
NPAD = 51200
STRIPE = NPAD // NS
CH = 256
SUB = 128
NSUB = CH // SUB
EPT = 51200
NCHUNK = EPT // CH
EPAD = EPT * NS
NSLOT = 4


def _spmm_kernel(sd_hbm, w_hbm, cur_hbm, out_hbm,
                 accum, sd_b, w_b, rows,
                 se0, se1, se2, se3, sg0, sg1, ss0, ss1, sw0, sw1):
    c = lax.axis_index("c")
    s = lax.axis_index("s")
    sems_e = [se0, se1, se2, se3]
    sems_g = [sg0, sg1]
    sems_s = [ss0, ss1]
    sems_w = [sw0, sw1]
    cur_view = cur_hbm.at[c]

    tbase_e = s * EPT
    tbase_r = s * (EPT // SUB)

    z = jnp.zeros((16,), jnp.float32)

    @pl.loop(0, CH, step=8)
    def _(r0):
        for i in range(8):
            rows[0, r0 + i, pl.ds(0, 16)] = z
            rows[0, r0 + i, pl.ds(16, 16)] = z

    stripe_base = s * STRIPE
    zsrc = rows.at[0]
    for q in range(STRIPE // CH):
        pltpu.sync_copy(zsrc, accum.at[pl.ds(stripe_base + q * CH, CH)])
    rem = STRIPE % CH
    if rem:
        pltpu.sync_copy(rows.at[0].at[pl.ds(0, rem)],
                        accum.at[pl.ds(stripe_base + (STRIPE // CH) * CH, rem)])

    plsc.subcore_barrier()

    def edata_start(g, slot):
        pltpu.make_async_copy(sd_hbm.at[pl.ds(tbase_r + NSUB * g, NSUB)],
                              sd_b.at[slot], sems_e[slot]).start()

    def edata_wait(slot):
        pltpu.make_async_copy(sd_hbm.at[pl.ds(0, NSUB)],
                              sd_b.at[slot], sems_e[slot]).wait()

    def wexp_start(g, ws):
        pltpu.make_async_copy(w_hbm.at[pl.ds(tbase_e + g * CH, CH)],
                              w_b.at[ws], sems_w[ws]).start()

    def wexp_wait(ws):
        pltpu.make_async_copy(w_hbm.at[pl.ds(0, CH)],
                              w_b.at[ws], sems_w[ws]).wait()

    def gather_start(slot, b):
        for j in range(NSUB):
            pltpu.make_async_copy(cur_view.at[sd_b.at[slot, j, 0]],
                                  rows.at[b].at[pl.ds(j * SUB, SUB)],
                                  sems_g[b]).start()

    def gather_wait(b):
        for j in range(NSUB):
            pltpu.make_async_copy(cur_view.at[pl.ds(0, SUB)],
                                  rows.at[b].at[pl.ds(j * SUB, SUB)],
                                  sems_g[b]).wait()

    def scatter_start(slot, b):
        for j in range(NSUB):
            pltpu.async_copy(rows.at[b].at[pl.ds(j * SUB, SUB)],
                             accum.at[sd_b.at[slot, j, 1]],
                             sems_s[b], add=True)

    def scatter_wait(b):
        for j in range(NSUB):
            pltpu.make_async_copy(cur_view.at[pl.ds(0, SUB)],
                                  rows.at[b].at[pl.ds(j * SUB, SUB)],
                                  sems_s[b]).wait()

    def multiply(ws, b):
        @plsc.parallel_loop(0, CH, step=8)
        def _(e0):
            for i in range(8):
                e = e0 + i
                w16 = w_b[ws, e, pl.ds(0, 16)]
                rows[b, e, pl.ds(0, 16)] = rows[b, e, pl.ds(0, 16)] * w16
                rows[b, e, pl.ds(16, 16)] = rows[b, e, pl.ds(16, 16)] * w16

    def iteration(g, m4, m2, wait_s=True, do_edata=True, do_gather=True):
        nb = (m2 + 1) % 2
        if wait_s:
            scatter_wait(nb)
        if do_edata:
            edata_start(g + 3, (m4 + 3) % 4)
        if do_gather:
            wexp_start(g + 1, nb)
            edata_wait((m4 + 1) % 4)
            gather_start((m4 + 1) % 4, nb)
        gather_wait(m2)
        wexp_wait(m2)
        multiply(m2, m2)
        scatter_start(m4, m2)

    for g in range(3):
        edata_start(g, g)
    wexp_start(0, 0)
    edata_wait(0)
    gather_start(0, 0)
    iteration(0, 0, 0, wait_s=False)

    @pl.loop(1, NCHUNK - 3, step=4)
    def _(g0):
        for i in range(4):
            iteration(g0 + i, (1 + i) % 4, (1 + i) % 2)

    iteration(NCHUNK - 3, (NCHUNK - 3) % 4, (NCHUNK - 3) % 2, do_edata=False)
    iteration(NCHUNK - 2, (NCHUNK - 2) % 4, (NCHUNK - 2) % 2, do_edata=False)
    iteration(NCHUNK - 1, (NCHUNK - 1) % 4, (NCHUNK - 1) % 2,
              do_edata=False, do_gather=False)
    scatter_wait((NCHUNK - 1) % 2)

    plsc.subcore_barrier()

    out_view = out_hbm.at[c]
    for q in range(STRIPE // CH):
        pltpu.sync_copy(accum.at[pl.ds(stripe_base + q * CH, CH)],
                        out_view.at[pl.ds(stripe_base + q * CH, CH)])
    if rem:
        base = stripe_base + (STRIPE // CH) * CH
        pltpu.sync_copy(accum.at[pl.ds(base, rem)], out_view.at[pl.ds(base, rem)])


_spmm = pl.kernel(
    _spmm_kernel,
    out_type=jax.ShapeDtypeStruct((NC, NPAD, HALF), jnp.float32),
    mesh=plsc.VectorSubcoreMesh(core_axis_name="c", subcore_axis_name="s"),
    scratch_types=[
        pltpu.VMEM_SHARED((NPAD, HALF), jnp.float32),
        pltpu.VMEM((NSLOT, NSUB, 2, SUB), jnp.int32),
        pltpu.VMEM((2, CH, 16), jnp.float32),
        pltpu.VMEM((2, CH, HALF), jnp.float32),
    ] + [pltpu.SemaphoreType.DMA] * 10,
    compiler_params=pltpu.CompilerParams(use_tc_tiling_on_sc=False,
                                         needs_layout_passes=False),
)


def _acc_body(out_ref, acc_ref, accn_ref):
    accn_ref[...] = acc_ref[...] + out_ref[...]


ROWS128 = NC * NPAD * HALF // 128
BLK = 3200

_acc_add = pl.pallas_call(
    _acc_body,
    out_shape=jax.ShapeDtypeStruct((ROWS128, 128), jnp.float32),
    grid=(ROWS128 // BLK,),
    in_specs=[
        pl.BlockSpec((BLK, 128), lambda i: (i, 0)),
        pl.BlockSpec((BLK, 128), lambda i: (i, 0)),
    ],
    out_specs=pl.BlockSpec((BLK, 128), lambda i: (i, 0)),
)


def kernel(edge_index, edge_weight, uEmbeds, iEmbeds):
    src = edge_index[0]
    dst = edge_index[1]
    pad = EPAD - E
    src2d = jnp.pad(src, (0, pad)).reshape(EPAD // SUB, SUB)
    dst2d = jnp.pad(dst, (0, pad), constant_values=N).reshape(EPAD // SUB, SUB)
    srcdst = jnp.stack([src2d, dst2d], axis=1)
    w1d = jnp.pad(edge_weight, (0, pad))
    wexp = jnp.broadcast_to(w1d[:, None], (EPAD, 16))

    embeds = jnp.concatenate([uEmbeds, iEmbeds], axis=0)
    rowpad = NPAD - N
    halves = jnp.stack([
        jnp.pad(embeds[:, :HALF], ((0, rowpad), (0, 0))),
        jnp.pad(embeds[:, HALF:], ((0, rowpad), (0, 0))),
    ])

    cur = halves
    acc = halves.reshape(ROWS128, 128)
    for _layer in range(N_LAYERS):
        cur = _spmm(srcdst, wexp, cur)
        acc = _acc_add(cur.reshape(ROWS128, 128), acc)

    acc = acc.reshape(NC, NPAD, HALF)
    full = jnp.concatenate([acc[0, :N], acc[1, :N]], axis=1)
    return full[:N_USER], full[N_USER:]


# speedup vs baseline: 1.4127x; 1.4127x over previous
import jax
import jax.numpy as jnp
from jax import lax
from jax.experimental import pallas as pl
from jax.experimental.pallas import tpu as pltpu
from jax.experimental.pallas import tpu_sc as plsc

N_USER = 25000
N_ITEM = 25000
N = N_USER + N_ITEM
E = 800000
D = 64
N_LAYERS = 3

NC = 2
NS = 16
HALF = 32
NPAD = 51200
STRIPE = NPAD // NS
CH = 256
EPT = 51200
NCHUNK = EPT // CH
EPAD = EPT * NS
NSLOT = 4


def _spmm_kernel(sd_hbm, cur_hbm, out_hbm,
                 accum, sd_b, rows,
                 se0, se1, se2, se3, sg0, sg1, ss0, ss1):
    c = lax.axis_index("c")
    s = lax.axis_index("s")
    sems_e = [se0, se1, se2, se3]
    sems_g = [sg0, sg1]
    sems_s = [ss0, ss1]
    cur_view = cur_hbm.at[c]

    tbase_c = s * NCHUNK

    z = jnp.zeros((16,), jnp.float32)

    @pl.loop(0, CH, step=8)
    def _(r0):
        for i in range(8):
            rows[0, r0 + i, pl.ds(0, 16)] = z
            rows[0, r0 + i, pl.ds(16, 16)] = z

    stripe_base = s * STRIPE
    zsrc = rows.at[0]
    for q in range(STRIPE // CH):
        pltpu.sync_copy(zsrc, accum.at[pl.ds(stripe_base + q * CH, CH)])
    rem = STRIPE % CH
    if rem:
        pltpu.sync_copy(rows.at[0].at[pl.ds(0, rem)],
                        accum.at[pl.ds(stripe_base + (STRIPE // CH) * CH, rem)])

    plsc.subcore_barrier()

    def edata_start(g, slot):
        pltpu.make_async_copy(sd_hbm.at[tbase_c + g], sd_b.at[slot],
                              sems_e[slot]).start()

    def edata_wait(slot):
        pltpu.make_async_copy(sd_hbm.at[0], sd_b.at[slot],
                              sems_e[slot]).wait()

    def gather_start(slot, b):
        pltpu.make_async_copy(cur_view.at[sd_b.at[slot, 0]], rows.at[b],
                              sems_g[b]).start()

    def gather_wait(b):
        pltpu.make_async_copy(cur_view.at[pl.ds(0, CH)], rows.at[b],
                              sems_g[b]).wait()

    def scatter_start(slot, b):
        pltpu.async_copy(rows.at[b], accum.at[sd_b.at[slot, 1]],
                         sems_s[b], add=True)

    def scatter_wait(b):
        pltpu.make_async_copy(cur_view.at[pl.ds(0, CH)], rows.at[b],
                              sems_s[b]).wait()

    def multiply(slot, b):
        wslot = sd_b.at[slot, 2]

        @plsc.parallel_loop(0, CH, step=8)
        def _(e0):
            for i in range(8):
                e = e0 + i
                w16i = plsc.load_gather(wslot, [jnp.full((16,), e, jnp.int32)])
                w16 = plsc.bitcast(w16i, jnp.float32)
                rows[b, e, pl.ds(0, 16)] = rows[b, e, pl.ds(0, 16)] * w16
                rows[b, e, pl.ds(16, 16)] = rows[b, e, pl.ds(16, 16)] * w16

    def iteration(g, m4, m2, wait_s=True, do_edata=True, do_gather=True):
        nb = (m2 + 1) % 2
        if wait_s:
            scatter_wait(nb)
        if do_edata:
            edata_start(g + 3, (m4 + 3) % 4)
        if do_gather:
            edata_wait((m4 + 1) % 4)
            gather_start((m4 + 1) % 4, nb)
        gather_wait(m2)
        multiply(m4, m2)
        scatter_start(m4, m2)

    for g in range(3):
        edata_start(g, g)
    edata_wait(0)
    gather_start(0, 0)
    iteration(0, 0, 0, wait_s=False)

    @pl.loop(1, NCHUNK - 3, step=4)
    def _(g0):
        for i in range(4):
            iteration(g0 + i, (1 + i) % 4, (1 + i) % 2)

    iteration(NCHUNK - 3, (NCHUNK - 3) % 4, (NCHUNK - 3) % 2, do_edata=False)
    iteration(NCHUNK - 2, (NCHUNK - 2) % 4, (NCHUNK - 2) % 2, do_edata=False)
    iteration(NCHUNK - 1, (NCHUNK - 1) % 4, (NCHUNK - 1) % 2,
              do_edata=False, do_gather=False)
    scatter_wait((NCHUNK - 1) % 2)

    plsc.subcore_barrier()

    out_view = out_hbm.at[c]
    for q in range(STRIPE // CH):
        pltpu.sync_copy(accum.at[pl.ds(stripe_base + q * CH, CH)],
                        out_view.at[pl.ds(stripe_base + q * CH, CH)])
    if rem:
        base = stripe_base + (STRIPE // CH) * CH
        pltpu.sync_copy(accum.at[pl.ds(base, rem)], out_view.at[pl.ds(base, rem)])


_spmm = pl.kernel(
    _spmm_kernel,
    out_type=jax.ShapeDtypeStruct((NC, NPAD, HALF), jnp.float32),
    mesh=plsc.VectorSubcoreMesh(core_axis_name="c", subcore_axis_name="s"),
    scratch_types=[
        pltpu.VMEM_SHARED((NPAD, HALF), jnp.float32),
        pltpu.VMEM((NSLOT, 3, CH), jnp.int32),
        pltpu.VMEM((2, CH, HALF), jnp.float32),
    ] + [pltpu.SemaphoreType.DMA] * 8,
    compiler_params=pltpu.CompilerParams(use_tc_tiling_on_sc=False,
                                         needs_layout_passes=False),
)


def _acc_body(out_ref, acc_ref, accn_ref):
    accn_ref[...] = acc_ref[...] + out_ref[...]


ROWS128 = NC * NPAD * HALF // 128
BLK = 3200

_acc_add = pl.pallas_call(
    _acc_body,
    out_shape=jax.ShapeDtypeStruct((ROWS128, 128), jnp.float32),
    grid=(ROWS128 // BLK,),
    in_specs=[
        pl.BlockSpec((BLK, 128), lambda i: (i, 0)),
        pl.BlockSpec((BLK, 128), lambda i: (i, 0)),
    ],
    out_specs=pl.BlockSpec((BLK, 128), lambda i: (i, 0)),
)


def kernel(edge_index, edge_weight, uEmbeds, iEmbeds):
    src = edge_index[0]
    dst = edge_index[1]
    pad = EPAD - E
    src2d = jnp.pad(src, (0, pad)).reshape(EPAD // CH, CH)
    dst2d = jnp.pad(dst, (0, pad), constant_values=N).reshape(EPAD // CH, CH)
    wbits = lax.bitcast_convert_type(jnp.pad(edge_weight, (0, pad)),
                                     jnp.int32).reshape(EPAD // CH, CH)
    srcdst = jnp.stack([src2d, dst2d, wbits], axis=1)

    embeds = jnp.concatenate([uEmbeds, iEmbeds], axis=0)
    rowpad = NPAD - N
    halves = jnp.stack([
        jnp.pad(embeds[:, :HALF], ((0, rowpad), (0, 0))),
        jnp.pad(embeds[:, HALF:], ((0, rowpad), (0, 0))),
    ])

    cur = halves
    acc = halves.reshape(ROWS128, 128)
    for _layer in range(N_LAYERS):
        cur = _spmm(srcdst, cur)
        acc = _acc_add(cur.reshape(ROWS128, 128), acc)

    acc = acc.reshape(NC, NPAD, HALF)
    full = jnp.concatenate([acc[0, :N], acc[1, :N]], axis=1)
    return full[:N_USER], full[N_USER:]


# speedup vs baseline: 1.4184x; 1.0040x over previous
import jax
import jax.numpy as jnp
from jax import lax
from jax.experimental import pallas as pl
from jax.experimental.pallas import tpu as pltpu
from jax.experimental.pallas import tpu_sc as plsc

N_USER = 25000
N_ITEM = 25000
N = N_USER + N_ITEM
E = 800000
D = 64
N_LAYERS = 3

NC = 2
NS = 16
HALF = 32
NPAD = 51200
STRIPE = NPAD // NS
CH = 256
EPT = 51200
NCHUNK = EPT // CH
EPAD = EPT * NS
NSLOT = 4


def _spmm_kernel(sd_hbm, cur_hbm, out_hbm,
                 accum, sd_b, rows,
                 se0, se1, se2, se3, sg0, sg1, ss0, ss1):
    c = lax.axis_index("c")
    s = lax.axis_index("s")
    sems_e = [se0, se1, se2, se3]
    sems_g = [sg0, sg1]
    sems_s = [ss0, ss1]
    cur_view = cur_hbm.at[c]

    tbase_c = s * NCHUNK

    z = jnp.zeros((16,), jnp.float32)

    @pl.loop(0, CH, step=8)
    def _(r0):
        for i in range(8):
            rows[0, r0 + i, pl.ds(0, 16)] = z
            rows[0, r0 + i, pl.ds(16, 16)] = z

    stripe_base = s * STRIPE
    zsrc = rows.at[0]
    zdescs = [
        pltpu.make_async_copy(zsrc, accum.at[pl.ds(stripe_base + q * CH, CH)],
                              sg0)
        for q in range(STRIPE // CH)
    ]
    rem = STRIPE % CH
    if rem:
        zdescs.append(pltpu.make_async_copy(
            rows.at[0].at[pl.ds(0, rem)],
            accum.at[pl.ds(stripe_base + (STRIPE // CH) * CH, rem)], sg0))
    for d in zdescs:
        d.start()
    for d in zdescs:
        d.wait()

    plsc.subcore_barrier()

    def edata_start(g, slot):
        pltpu.make_async_copy(sd_hbm.at[tbase_c + g], sd_b.at[slot],
                              sems_e[slot]).start()

    def edata_wait(slot):
        pltpu.make_async_copy(sd_hbm.at[0], sd_b.at[slot],
                              sems_e[slot]).wait()

    def gather_start(slot, b):
        pltpu.make_async_copy(cur_view.at[sd_b.at[slot, 0]], rows.at[b],
                              sems_g[b]).start()

    def gather_wait(b):
        pltpu.make_async_copy(cur_view.at[pl.ds(0, CH)], rows.at[b],
                              sems_g[b]).wait()

    def scatter_start(slot, b):
        pltpu.async_copy(rows.at[b], accum.at[sd_b.at[slot, 1]],
                         sems_s[b], add=True)

    def scatter_wait(b):
        pltpu.make_async_copy(cur_view.at[pl.ds(0, CH)], rows.at[b],
                              sems_s[b]).wait()

    def multiply(slot, b):
        wslot = sd_b.at[slot, 2]

        @plsc.parallel_loop(0, CH, step=8)
        def _(e0):
            for i in range(8):
                e = e0 + i
                w16i = plsc.load_gather(wslot, [jnp.full((16,), e, jnp.int32)])
                w16 = plsc.bitcast(w16i, jnp.float32)
                rows[b, e, pl.ds(0, 16)] = rows[b, e, pl.ds(0, 16)] * w16
                rows[b, e, pl.ds(16, 16)] = rows[b, e, pl.ds(16, 16)] * w16

    def iteration(g, m4, m2, wait_s=True, do_edata=True, do_gather=True):
        nb = (m2 + 1) % 2
        if wait_s:
            scatter_wait(nb)
        if do_edata:
            edata_start(g + 3, (m4 + 3) % 4)
        if do_gather:
            edata_wait((m4 + 1) % 4)
            gather_start((m4 + 1) % 4, nb)
        gather_wait(m2)
        multiply(m4, m2)
        scatter_start(m4, m2)

    EDGE_LOOP = True
    if EDGE_LOOP:
      for g in range(3):
        edata_start(g, g)
    if EDGE_LOOP:
      edata_wait(0)
      gather_start(0, 0)
      iteration(0, 0, 0, wait_s=False)

      @pl.loop(1, NCHUNK - 3, step=4)
      def _(g0):
        for i in range(4):
            iteration(g0 + i, (1 + i) % 4, (1 + i) % 2)

      iteration(NCHUNK - 3, (NCHUNK - 3) % 4, (NCHUNK - 3) % 2, do_edata=False)
      iteration(NCHUNK - 2, (NCHUNK - 2) % 4, (NCHUNK - 2) % 2, do_edata=False)
      iteration(NCHUNK - 1, (NCHUNK - 1) % 4, (NCHUNK - 1) % 2,
                do_edata=False, do_gather=False)
      scatter_wait((NCHUNK - 1) % 2)

    plsc.subcore_barrier()

    out_view = out_hbm.at[c]
    ddescs = [
        pltpu.make_async_copy(accum.at[pl.ds(stripe_base + q * CH, CH)],
                              out_view.at[pl.ds(stripe_base + q * CH, CH)], sg0)
        for q in range(STRIPE // CH)
    ]
    if rem:
        base = stripe_base + (STRIPE // CH) * CH
        ddescs.append(pltpu.make_async_copy(
            accum.at[pl.ds(base, rem)], out_view.at[pl.ds(base, rem)], sg0))
    for d in ddescs:
        d.start()
    for d in ddescs:
        d.wait()


_spmm = pl.kernel(
    _spmm_kernel,
    out_type=jax.ShapeDtypeStruct((NC, NPAD, HALF), jnp.float32),
    mesh=plsc.VectorSubcoreMesh(core_axis_name="c", subcore_axis_name="s"),
    scratch_types=[
        pltpu.VMEM_SHARED((NPAD, HALF), jnp.float32),
        pltpu.VMEM((NSLOT, 3, CH), jnp.int32),
        pltpu.VMEM((2, CH, HALF), jnp.float32),
    ] + [pltpu.SemaphoreType.DMA] * 8,
    compiler_params=pltpu.CompilerParams(use_tc_tiling_on_sc=False,
                                         needs_layout_passes=False),
)


def _acc_body(out_ref, acc_ref, accn_ref):
    accn_ref[...] = acc_ref[...] + out_ref[...]


ROWS128 = NC * NPAD * HALF // 128
BLK = 3200

_acc_add = pl.pallas_call(
    _acc_body,
    out_shape=jax.ShapeDtypeStruct((ROWS128, 128), jnp.float32),
    grid=(ROWS128 // BLK,),
    in_specs=[
        pl.BlockSpec((BLK, 128), lambda i: (i, 0)),
        pl.BlockSpec((BLK, 128), lambda i: (i, 0)),
    ],
    out_specs=pl.BlockSpec((BLK, 128), lambda i: (i, 0)),
)


def kernel(edge_index, edge_weight, uEmbeds, iEmbeds):
    src = edge_index[0]
    dst = edge_index[1]
    pad = EPAD - E
    src2d = jnp.pad(src, (0, pad)).reshape(EPAD // CH, CH)
    dst2d = jnp.pad(dst, (0, pad), constant_values=N).reshape(EPAD // CH, CH)
    wbits = lax.bitcast_convert_type(jnp.pad(edge_weight, (0, pad)),
                                     jnp.int32).reshape(EPAD // CH, CH)
    srcdst = jnp.stack([src2d, dst2d, wbits], axis=1)

    embeds = jnp.concatenate([uEmbeds, iEmbeds], axis=0)
    rowpad = NPAD - N
    halves = jnp.stack([
        jnp.pad(embeds[:, :HALF], ((0, rowpad), (0, 0))),
        jnp.pad(embeds[:, HALF:], ((0, rowpad), (0, 0))),
    ])

    cur = halves
    acc = halves.reshape(ROWS128, 128)
    for _layer in range(N_LAYERS):
        cur = _spmm(srcdst, cur)
        acc = _acc_add(cur.reshape(ROWS128, 128), acc)

    acc = acc.reshape(NC, NPAD, HALF)
    full = jnp.concatenate([acc[0, :N], acc[1, :N]], axis=1)
    return full[:N_USER], full[N_USER:]


# speedup vs baseline: 1.4338x; 1.0108x over previous
import jax
import jax.numpy as jnp
from jax import lax
from jax.experimental import pallas as pl
from jax.experimental.pallas import tpu as pltpu
from jax.experimental.pallas import tpu_sc as plsc

N_USER = 25000
N_ITEM = 25000
N = N_USER + N_ITEM
E = 800000
D = 64
N_LAYERS = 3

NC = 2
NS = 16
HALF = 32
NPAD = 51200
STRIPE = NPAD // NS
CH = 256
EPT = 51200
NCHUNK = EPT // CH
EPAD = EPT * NS
NSLOT = 4


def _spmm_kernel(sd_hbm, cur_hbm, out_hbm,
                 accum, sd_b, rows, rows_bf,
                 se0, se1, se2, se3, sg0, sg1, ss0, ss1):
    c = lax.axis_index("c")
    s = lax.axis_index("s")
    sems_e = [se0, se1, se2, se3]
    sems_g = [sg0, sg1]
    sems_s = [ss0, ss1]
    cur_view = cur_hbm.at[c]
    out_view = out_hbm.at[c]

    tbase_c = s * NCHUNK

    z = jnp.zeros((16,), jnp.float32)

    @pl.loop(0, CH, step=8)
    def _(r0):
        for i in range(8):
            rows[0, r0 + i, pl.ds(0, 16)] = z
            rows[0, r0 + i, pl.ds(16, 16)] = z

    stripe_base = s * STRIPE
    zsrc = rows.at[0]
    zdescs = [
        pltpu.make_async_copy(zsrc, accum.at[pl.ds(stripe_base + q * CH, CH)],
                              sg0)
        for q in range(STRIPE // CH)
    ]
    rem = STRIPE % CH
    if rem:
        zdescs.append(pltpu.make_async_copy(
            rows.at[0].at[pl.ds(0, rem)],
            accum.at[pl.ds(stripe_base + (STRIPE // CH) * CH, rem)], sg0))
    for d in zdescs:
        d.start()
    for d in zdescs:
        d.wait()

    plsc.subcore_barrier()

    def edata_start(g, slot):
        pltpu.make_async_copy(sd_hbm.at[tbase_c + g], sd_b.at[slot],
                              sems_e[slot]).start()

    def edata_wait(slot):
        pltpu.make_async_copy(sd_hbm.at[0], sd_b.at[slot],
                              sems_e[slot]).wait()

    def gather_start(slot, b):
        pltpu.make_async_copy(cur_view.at[sd_b.at[slot, 0]], rows_bf.at[b],
                              sems_g[b]).start()

    def gather_wait(b):
        pltpu.make_async_copy(cur_view.at[pl.ds(0, CH)], rows_bf.at[b],
                              sems_g[b]).wait()

    def scatter_start(slot, b):
        pltpu.async_copy(rows.at[b], accum.at[sd_b.at[slot, 1]],
                         sems_s[b], add=True)

    def scatter_wait(b):
        pltpu.make_async_copy(out_view.at[pl.ds(0, CH)], rows.at[b],
                              sems_s[b]).wait()

    def multiply(slot, b):
        wslot = sd_b.at[slot, 2]

        @plsc.parallel_loop(0, CH, step=8)
        def _(e0):
            for i in range(8):
                e = e0 + i
                w16i = plsc.load_gather(wslot, [jnp.full((16,), e, jnp.int32)])
                w16 = plsc.bitcast(w16i, jnp.float32)
                v32 = rows_bf[b, e, pl.ds(0, 32)]
                lo, hi = plsc.unpack(v32, format=plsc.PackFormat.INTERLEAVED,
                                     preferred_element_type=jnp.float32)
                rows[b, e, pl.ds(0, 16)] = lo * w16
                rows[b, e, pl.ds(16, 16)] = hi * w16

    def iteration(g, m4, m2, wait_s=True, do_edata=True, do_gather=True):
        nb = (m2 + 1) % 2
        if wait_s:
            scatter_wait(nb)
        if do_edata:
            edata_start(g + 3, (m4 + 3) % 4)
        if do_gather:
            edata_wait((m4 + 1) % 4)
            gather_start((m4 + 1) % 4, nb)
        gather_wait(m2)
        multiply(m4, m2)
        scatter_start(m4, m2)

    EDGE_LOOP = True
    if EDGE_LOOP:
      for g in range(3):
        edata_start(g, g)
    if EDGE_LOOP:
      edata_wait(0)
      gather_start(0, 0)
      iteration(0, 0, 0, wait_s=False)

      @pl.loop(1, NCHUNK - 3, step=4)
      def _(g0):
        for i in range(4):
            iteration(g0 + i, (1 + i) % 4, (1 + i) % 2)

      iteration(NCHUNK - 3, (NCHUNK - 3) % 4, (NCHUNK - 3) % 2, do_edata=False)
      iteration(NCHUNK - 2, (NCHUNK - 2) % 4, (NCHUNK - 2) % 2, do_edata=False)
      iteration(NCHUNK - 1, (NCHUNK - 1) % 4, (NCHUNK - 1) % 2,
                do_edata=False, do_gather=False)
      scatter_wait((NCHUNK - 1) % 2)

    plsc.subcore_barrier()

    ddescs = [
        pltpu.make_async_copy(accum.at[pl.ds(stripe_base + q * CH, CH)],
                              out_view.at[pl.ds(stripe_base + q * CH, CH)], sg0)
        for q in range(STRIPE // CH)
    ]
    if rem:
        base = stripe_base + (STRIPE // CH) * CH
        ddescs.append(pltpu.make_async_copy(
            accum.at[pl.ds(base, rem)], out_view.at[pl.ds(base, rem)], sg0))
    for d in ddescs:
        d.start()
    for d in ddescs:
        d.wait()


_spmm = pl.kernel(
    _spmm_kernel,
    out_type=jax.ShapeDtypeStruct((NC, NPAD, HALF), jnp.float32),
    mesh=plsc.VectorSubcoreMesh(core_axis_name="c", subcore_axis_name="s"),
    scratch_types=[
        pltpu.VMEM_SHARED((NPAD, HALF), jnp.float32),
        pltpu.VMEM((NSLOT, 3, CH), jnp.int32),
        pltpu.VMEM((2, CH, HALF), jnp.float32),
        pltpu.VMEM((2, CH, HALF), jnp.bfloat16),
    ] + [pltpu.SemaphoreType.DMA] * 8,
    compiler_params=pltpu.CompilerParams(use_tc_tiling_on_sc=False,
                                         needs_layout_passes=False),
)


def _interleave_cols(x):
    shape = x.shape
    y = x.reshape(shape[:-1] + (shape[-1] // 32, 2, 16))
    y = jnp.swapaxes(y, -1, -2)
    return y.reshape(shape)


def _acc_body(out_ref, acc_ref, accn_ref):
    accn_ref[...] = acc_ref[...] + out_ref[...]


ROWS128 = NC * NPAD * HALF // 128
BLK = 3200

_acc_add = pl.pallas_call(
    _acc_body,
    out_shape=jax.ShapeDtypeStruct((ROWS128, 128), jnp.float32),
    grid=(ROWS128 // BLK,),
    in_specs=[
        pl.BlockSpec((BLK, 128), lambda i: (i, 0)),
        pl.BlockSpec((BLK, 128), lambda i: (i, 0)),
    ],
    out_specs=pl.BlockSpec((BLK, 128), lambda i: (i, 0)),
)


def kernel(edge_index, edge_weight, uEmbeds, iEmbeds):
    src = edge_index[0]
    dst = edge_index[1]
    pad = EPAD - E
    src2d = jnp.pad(src, (0, pad)).reshape(EPAD // CH, CH)
    dst2d = jnp.pad(dst, (0, pad), constant_values=N).reshape(EPAD // CH, CH)
    wbits = lax.bitcast_convert_type(jnp.pad(edge_weight, (0, pad)),
                                     jnp.int32).reshape(EPAD // CH, CH)
    srcdst = jnp.stack([src2d, dst2d, wbits], axis=1)

    embeds = jnp.concatenate([uEmbeds, iEmbeds], axis=0)
    rowpad = NPAD - N
    halves = jnp.stack([
        jnp.pad(embeds[:, :HALF], ((0, rowpad), (0, 0))),
        jnp.pad(embeds[:, HALF:], ((0, rowpad), (0, 0))),
    ])

    cur_bf = _interleave_cols(halves.astype(jnp.bfloat16))
    acc = halves.reshape(ROWS128, 128)
    for _layer in range(N_LAYERS):
        out = _spmm(srcdst, cur_bf)
        acc = _acc_add(out.reshape(ROWS128, 128), acc)
        cur_bf = _interleave_cols(out.astype(jnp.bfloat16))

    acc = acc.reshape(NC, NPAD, HALF)
    full = jnp.concatenate([acc[0, :N], acc[1, :N]], axis=1)
    return full[:N_USER], full[N_USER:]
